# baseline (speedup 1.0000x reference)
import math

import jax
import jax.numpy as jnp
from jax.experimental import pallas as pl
from jax.experimental.pallas import tpu as pltpu

_NUM_PROJS = 8


def _make_body(k, d, n):
    def _lsh_body(xt_ref, pdt_ref, w2t_ref, ptt_ref, one_ref, out_ref):
        pdt = pdt_ref[...]
        w2t = w2t_ref[...]
        ptt = ptt_ref[...]
        one = one_ref[...]
        for s in range(k):
            xt = xt_ref[pl.ds(s * d, d), :]
            yt = jnp.dot(pdt, xt, preferred_element_type=jnp.float32)
            pm = jnp.where(yt > 0.0, 1.0, -1.0).astype(jnp.bfloat16)
            a = jnp.dot(w2t, pm, preferred_element_type=jnp.float32)
            oh = jnp.where(a == 4.0, 1.0, 0.0).astype(jnp.bfloat16)
            rows = jnp.dot(ptt, oh[16:32, :],
                           preferred_element_type=jnp.float32
                           ).astype(jnp.bfloat16)
            vals = jnp.dot(one, oh[0:16, :] * rows,
                           preferred_element_type=jnp.float32)
            ids = vals.astype(jnp.int32).reshape(n)
            out_ref[pl.ds(s * (n // 128), n // 128), :] = ids.reshape(
                n // 128, 128)
    return _lsh_body


def kernel(mat, proj_dir, perm):
    b, h, n, d = mat.shape
    m = b * h * n
    bh = b * h
    xt = mat.transpose(0, 1, 3, 2).reshape(bh * d, n)
    pdt = proj_dir.reshape(d, _NUM_PROJS).astype(jnp.float32).T

    nib = jnp.arange(16, dtype=jnp.int32)
    hb = (2 * ((nib[None, :] >> jnp.arange(4, dtype=jnp.int32)[:, None]) & 1)
          - 1).astype(jnp.float32)
    zeros4 = jnp.zeros((4, 16), jnp.float32)
    w_lo = jnp.concatenate([hb, zeros4], axis=0)
    w_hi = jnp.concatenate([zeros4, hb], axis=0)
    w2t = jnp.concatenate([w_lo.T, w_hi.T], axis=0).astype(jnp.bfloat16)
    ptt = perm.reshape(16, 16).astype(jnp.bfloat16).T
    one = jnp.ones((1, 16), jnp.bfloat16)

    k = math.gcd(bh, 8)
    grid = bh // k

    out = pl.pallas_call(
        _make_body(k, d, n),
        grid=(grid,),
        in_specs=[
            pl.BlockSpec((k * d, n), lambda i: (i, 0)),
            pl.BlockSpec((_NUM_PROJS, d), lambda i: (0, 0)),
            pl.BlockSpec((32, _NUM_PROJS), lambda i: (0, 0)),
            pl.BlockSpec((16, 16), lambda i: (0, 0)),
            pl.BlockSpec((1, 16), lambda i: (0, 0)),
        ],
        out_specs=pl.BlockSpec((k * n // 128, 128), lambda i: (i, 0)),
        out_shape=jax.ShapeDtypeStruct((m // 128, 128), jnp.int32),
        compiler_params=pltpu.CompilerParams(
            dimension_semantics=("arbitrary",),
        ),
    )(xt, pdt, w2t, ptt, one)
    return out.reshape(b, h, n)
